# baseline (speedup 1.0000x reference)
import functools

import jax
import jax.numpy as jnp
from jax.experimental import pallas as pl
from jax.experimental.pallas import tpu as pltpu

N_TOK = 4096
D = 768
E = 8
BLK = 1024
NB = N_TOK // BLK


def _cv2(v):
    m = jnp.mean(v)
    var = jnp.mean((v - m) ** 2)
    return var / (m * m + 1e-10)


def _moe_body(x_ref, wg_ref, w_ref, b_ref, out_ref, loss_ref, imp_ref,
              load_ref, wb_ref, w32_ref, sems):
    i = pl.program_id(0)

    def _w_copy(e, buf):
        return pltpu.make_async_copy(
            w_ref.at[pl.ds(e * D, D)], w32_ref.at[buf], sems.at[buf])

    @pl.when(i == 0)
    def _start_w():
        _w_copy(0, 0).start()
        _w_copy(1, 1).start()

    xf = x_ref[...]
    logits = jnp.dot(xf, wg_ref[...], preferred_element_type=jnp.float32)

    e_iota = jax.lax.broadcasted_iota(jnp.int32, logits.shape, 1)
    v1 = jnp.max(logits, axis=1, keepdims=True)
    idx1 = jnp.min(jnp.where(logits == v1, e_iota, E), axis=1, keepdims=True)
    masked = jnp.where(e_iota == idx1, -jnp.inf, logits)
    v2 = jnp.max(masked, axis=1, keepdims=True)
    idx2 = jnp.min(jnp.where(masked == v2, e_iota, E), axis=1, keepdims=True)

    e2 = jnp.exp(v2 - v1)
    denom = 1.0 + e2
    g1 = 1.0 / denom
    g2 = e2 / denom
    gates = (jnp.where(e_iota == idx1, g1, 0.0)
             + jnp.where(e_iota == idx2, g2, 0.0))

    acc = jnp.dot(gates, b_ref[...], preferred_element_type=jnp.float32)
    xb = xf.astype(jnp.bfloat16)

    @pl.when(i == 0)
    def _first_step():
        a = acc
        for e in range(E):
            buf = e % 2
            _w_copy(e, buf).wait()
            wb_ref[pl.ds(e * D, D)] = w32_ref[buf].astype(jnp.bfloat16)
            if e + 2 < E:
                _w_copy(e + 2, buf).start()
            ye = jnp.dot(xb, wb_ref[pl.ds(e * D, D)],
                         preferred_element_type=jnp.float32)
            a += gates[:, e:e + 1] * ye
        out_ref[...] = a

    @pl.when(i > 0)
    def _steady():
        a = acc
        for e in range(E):
            ye = jnp.dot(xb, wb_ref[pl.ds(e * D, D)],
                         preferred_element_type=jnp.float32)
            a += gates[:, e:e + 1] * ye
        out_ref[...] = a

    imp_blk = jnp.sum(gates, axis=0, keepdims=True)
    load_blk = jnp.sum((gates > 0.0).astype(jnp.float32), axis=0, keepdims=True)

    @pl.when(i == 0)
    def _init():
        imp_ref[...] = jnp.zeros_like(imp_ref)
        load_ref[...] = jnp.zeros_like(load_ref)

    imp_ref[...] += imp_blk
    load_ref[...] += load_blk

    @pl.when(i == pl.num_programs(0) - 1)
    def _fin():
        loss = _cv2(imp_ref[...]) + _cv2(load_ref[...])
        loss_ref[...] = jnp.broadcast_to(loss, (1, 1))


@jax.jit
def kernel(x, Wg, W, b):
    wb = W.reshape(E * D, D)
    out, loss = pl.pallas_call(
        _moe_body,
        grid=(NB,),
        in_specs=[
            pl.BlockSpec((BLK, D), lambda i: (i, 0)),
            pl.BlockSpec((D, E), lambda i: (0, 0)),
            pl.BlockSpec(memory_space=pl.ANY),
            pl.BlockSpec((E, D), lambda i: (0, 0)),
        ],
        out_specs=[
            pl.BlockSpec((BLK, D), lambda i: (i, 0)),
            pl.BlockSpec((1, 1), lambda i: (0, 0)),
        ],
        out_shape=[
            jax.ShapeDtypeStruct((N_TOK, D), jnp.float32),
            jax.ShapeDtypeStruct((1, 1), jnp.float32),
        ],
        scratch_shapes=[
            pltpu.VMEM((1, E), jnp.float32),
            pltpu.VMEM((1, E), jnp.float32),
            pltpu.VMEM((E * D, D), jnp.bfloat16),
            pltpu.VMEM((2, D, D), jnp.float32),
            pltpu.SemaphoreType.DMA((2,)),
        ],
    )(x, Wg, wb, b)
    return out, loss.reshape(())


# speedup vs baseline: 1.0376x; 1.0376x over previous
import functools

import jax
import jax.numpy as jnp
from jax.experimental import pallas as pl
from jax.experimental.pallas import tpu as pltpu

N_TOK = 4096
D = 768
E = 8
BLK = 512
NB = N_TOK // BLK


def _cv2(v):
    m = jnp.mean(v)
    var = jnp.mean((v - m) ** 2)
    return var / (m * m + 1e-10)


def _moe_body(x_ref, wg_ref, w_ref, b_ref, out_ref, loss_ref, imp_ref,
              load_ref, wb_ref):
    i = pl.program_id(0)

    @pl.when(i == 0)
    def _cast_w():
        wb_ref[...] = w_ref[...].astype(jnp.bfloat16)

    xf = x_ref[...]
    logits = jnp.dot(xf, wg_ref[...], preferred_element_type=jnp.float32)

    e_iota = jax.lax.broadcasted_iota(jnp.int32, logits.shape, 1)
    v1 = jnp.max(logits, axis=1, keepdims=True)
    idx1 = jnp.min(jnp.where(logits == v1, e_iota, E), axis=1, keepdims=True)
    masked = jnp.where(e_iota == idx1, -jnp.inf, logits)
    v2 = jnp.max(masked, axis=1, keepdims=True)
    idx2 = jnp.min(jnp.where(masked == v2, e_iota, E), axis=1, keepdims=True)

    e2 = jnp.exp(v2 - v1)
    denom = 1.0 + e2
    g1 = 1.0 / denom
    g2 = e2 / denom
    gates = (jnp.where(e_iota == idx1, g1, 0.0)
             + jnp.where(e_iota == idx2, g2, 0.0))

    acc = jnp.dot(gates, b_ref[...], preferred_element_type=jnp.float32)
    xb = xf.astype(jnp.bfloat16)
    for e in range(E):
        ye = jnp.dot(xb, wb_ref[pl.ds(e * D, D)],
                     preferred_element_type=jnp.float32)
        acc += gates[:, e:e + 1] * ye
    out_ref[...] = acc

    imp_blk = jnp.sum(gates, axis=0, keepdims=True)
    load_blk = jnp.sum((gates > 0.0).astype(jnp.float32), axis=0, keepdims=True)

    @pl.when(i == 0)
    def _init():
        imp_ref[...] = jnp.zeros_like(imp_ref)
        load_ref[...] = jnp.zeros_like(load_ref)

    imp_ref[...] += imp_blk
    load_ref[...] += load_blk

    @pl.when(i == pl.num_programs(0) - 1)
    def _fin():
        loss = _cv2(imp_ref[...]) + _cv2(load_ref[...])
        loss_ref[...] = jnp.broadcast_to(loss, (1, 1))


@jax.jit
def kernel(x, Wg, W, b):
    wb = W.reshape(E * D, D)
    out, loss = pl.pallas_call(
        _moe_body,
        grid=(NB,),
        in_specs=[
            pl.BlockSpec((BLK, D), lambda i: (i, 0)),
            pl.BlockSpec((D, E), lambda i: (0, 0)),
            pl.BlockSpec((E * D, D), lambda i: (0, 0)),
            pl.BlockSpec((E, D), lambda i: (0, 0)),
        ],
        out_specs=[
            pl.BlockSpec((BLK, D), lambda i: (i, 0)),
            pl.BlockSpec((1, 1), lambda i: (0, 0)),
        ],
        out_shape=[
            jax.ShapeDtypeStruct((N_TOK, D), jnp.float32),
            jax.ShapeDtypeStruct((1, 1), jnp.float32),
        ],
        scratch_shapes=[
            pltpu.VMEM((1, E), jnp.float32),
            pltpu.VMEM((1, E), jnp.float32),
            pltpu.VMEM((E * D, D), jnp.bfloat16),
        ],
    )(x, Wg, wb, b)
    return out, loss.reshape(())


# speedup vs baseline: 1.0764x; 1.0373x over previous
import functools

import jax
import jax.numpy as jnp
from jax.experimental import pallas as pl
from jax.experimental.pallas import tpu as pltpu

N_TOK = 4096
D = 768
E = 8
BLK = 1024
NB = N_TOK // BLK


def _cv2(v):
    m = jnp.mean(v)
    var = jnp.mean((v - m) ** 2)
    return var / (m * m + 1e-10)


def _moe_body(x_ref, wg_ref, w_ref, b_ref, out_ref, loss_ref, imp_ref,
              load_ref, wb_ref):
    i = pl.program_id(0)

    @pl.when(i == 0)
    def _cast_w():
        wb_ref[...] = w_ref[...].astype(jnp.bfloat16)

    xf = x_ref[...]
    logits = jnp.dot(xf, wg_ref[...], preferred_element_type=jnp.float32)

    e_iota = jax.lax.broadcasted_iota(jnp.int32, logits.shape, 1)
    v1 = jnp.max(logits, axis=1, keepdims=True)
    idx1 = jnp.min(jnp.where(logits == v1, e_iota, E), axis=1, keepdims=True)
    masked = jnp.where(e_iota == idx1, -jnp.inf, logits)
    v2 = jnp.max(masked, axis=1, keepdims=True)
    idx2 = jnp.min(jnp.where(masked == v2, e_iota, E), axis=1, keepdims=True)

    e2 = jnp.exp(v2 - v1)
    denom = 1.0 + e2
    g1 = 1.0 / denom
    g2 = e2 / denom
    gates = (jnp.where(e_iota == idx1, g1, 0.0)
             + jnp.where(e_iota == idx2, g2, 0.0))

    acc = jnp.dot(gates, b_ref[...], preferred_element_type=jnp.float32)
    xb = xf.astype(jnp.bfloat16)
    for e in range(E):
        ye = jnp.dot(xb, wb_ref[pl.ds(e * D, D)],
                     preferred_element_type=jnp.float32)
        acc += gates[:, e:e + 1] * ye
    out_ref[...] = acc

    imp_blk = jnp.sum(gates, axis=0, keepdims=True)
    load_blk = jnp.sum((gates > 0.0).astype(jnp.float32), axis=0, keepdims=True)

    @pl.when(i == 0)
    def _init():
        imp_ref[...] = jnp.zeros_like(imp_ref)
        load_ref[...] = jnp.zeros_like(load_ref)

    imp_ref[...] += imp_blk
    load_ref[...] += load_blk

    @pl.when(i == pl.num_programs(0) - 1)
    def _fin():
        loss = _cv2(imp_ref[...]) + _cv2(load_ref[...])
        loss_ref[...] = jnp.broadcast_to(loss, (1, 1))


@jax.jit
def kernel(x, Wg, W, b):
    wb = W.reshape(E * D, D)
    out, loss = pl.pallas_call(
        _moe_body,
        grid=(NB,),
        in_specs=[
            pl.BlockSpec((BLK, D), lambda i: (i, 0)),
            pl.BlockSpec((D, E), lambda i: (0, 0)),
            pl.BlockSpec((E * D, D), lambda i: (0, 0)),
            pl.BlockSpec((E, D), lambda i: (0, 0)),
        ],
        out_specs=[
            pl.BlockSpec((BLK, D), lambda i: (i, 0)),
            pl.BlockSpec((1, 1), lambda i: (0, 0)),
        ],
        out_shape=[
            jax.ShapeDtypeStruct((N_TOK, D), jnp.float32),
            jax.ShapeDtypeStruct((1, 1), jnp.float32),
        ],
        scratch_shapes=[
            pltpu.VMEM((1, E), jnp.float32),
            pltpu.VMEM((1, E), jnp.float32),
            pltpu.VMEM((E * D, D), jnp.bfloat16),
        ],
    )(x, Wg, wb, b)
    return out, loss.reshape(())
